# baseline (speedup 1.0000x reference)
import functools
import math

import jax
import jax.numpy as jnp
from jax import lax
from jax.experimental import pallas as pl
from jax.experimental.pallas import tpu as pltpu
from jax.experimental.pallas import tpu_sc as plsc

D_MODEL = 64
SCALE = math.sqrt(D_MODEL)

NBUF = 6
LAG = 3


@functools.lru_cache(maxsize=None)
def _make_gather(s, p, v, d):
    info = plsc.get_sparse_core_info()
    nc, ns = info.num_cores, info.num_subcores
    nw = nc * ns
    assert s % nw == 0 and p % 2 == 0
    srows = s // nw
    h0 = min(128, (p // 2 + 7) // 8 * 8)
    splits = [(0, h0), (h0, p - h0)]
    assert all(ln <= 128 and ln % 8 == 0 and off % 8 == 0
               for off, ln in splits)
    b = s * p

    mesh = plsc.VectorSubcoreMesh(core_axis_name="c", subcore_axis_name="s")

    @functools.partial(
        pl.kernel,
        mesh=mesh,
        compiler_params=pltpu.CompilerParams(use_tc_tiling_on_sc=False),
        out_type=jax.ShapeDtypeStruct((b, d), jnp.float32),
        scratch_types=[
            pltpu.VMEM((srows, p), jnp.int32),
            pltpu.VMEM((NBUF, p, d), jnp.float32),
        ]
        + [pltpu.SemaphoreType.DMA] * (2 * NBUF),
    )
    def gather_kernel(idx_hbm, tab_hbm, out_hbm, idx_v, rows_v, *sems):
        gsems = sems[:NBUF]
        osems = sems[NBUF:]
        wid = lax.axis_index("s") * nc + lax.axis_index("c")
        row_base = wid * srows * p

        pltpu.sync_copy(idx_hbm.at[pl.ds(wid * srows, srows)], idx_v)

        def gather_descs(t):
            bslot = t % NBUF
            return [
                (
                    tab_hbm.at[idx_v.at[t, pl.ds(off, ln)]],
                    rows_v.at[bslot, pl.ds(off, ln)],
                    gsems[bslot],
                )
                for off, ln in splits
            ]

        def write_desc(t):
            bslot = t % NBUF
            return (
                rows_v.at[bslot],
                out_hbm.at[pl.ds(row_base + t * p, p)],
                osems[bslot],
            )

        for t in range(srows + LAG):
            if t < srows:
                if t >= NBUF:
                    pltpu.make_async_copy(*write_desc(t - NBUF)).wait()
                for desc in gather_descs(t):
                    pltpu.async_copy(*desc)
            tp = t - LAG
            if tp >= 0:
                for desc in gather_descs(tp):
                    pltpu.make_async_copy(*desc).wait()
                pltpu.async_copy(*write_desc(tp))

        for t in range(srows + LAG - NBUF, srows):
            pltpu.make_async_copy(*write_desc(t)).wait()

    return gather_kernel


def kernel(x, table):
    v, d = table.shape
    s, p = x.shape
    xi = x.astype(jnp.int32)
    interm = _make_gather(s, p, v, d)(xi, table)
    return (interm * SCALE).reshape(s, p, d)


# speedup vs baseline: 1.3625x; 1.3625x over previous
import functools
import math

import jax
import jax.numpy as jnp
from jax import lax
from jax.experimental import pallas as pl
from jax.experimental.pallas import tpu as pltpu
from jax.experimental.pallas import tpu_sc as plsc

D_MODEL = 64
SCALE = math.sqrt(D_MODEL)

NBUF = 6
LAG = 3
LANES = 16


@functools.lru_cache(maxsize=None)
def _make_gather(s, p, v, d):
    info = plsc.get_sparse_core_info()
    nc, ns = info.num_cores, info.num_subcores
    nw = nc * ns
    assert s % nw == 0 and d % LANES == 0
    srows = s // nw
    h0 = min(128, (p // 2 + 7) // 8 * 8)
    splits = [(0, h0), (h0, p - h0)]
    assert all(0 < ln <= 128 and ln % 8 == 0 and off % 8 == 0
               for off, ln in splits)
    rows_per_iter = 4
    assert p % rows_per_iter == 0

    mesh = plsc.VectorSubcoreMesh(core_axis_name="c", subcore_axis_name="s")

    @functools.partial(
        pl.kernel,
        mesh=mesh,
        compiler_params=pltpu.CompilerParams(use_tc_tiling_on_sc=False),
        out_type=jax.ShapeDtypeStruct((s, p, d), jnp.float32),
        scratch_types=[
            pltpu.VMEM((srows, p), jnp.int32),
            pltpu.VMEM((NBUF, p, d), jnp.float32),
        ]
        + [pltpu.SemaphoreType.DMA] * (2 * NBUF),
    )
    def gather_kernel(idx_hbm, tab_hbm, out_hbm, idx_v, rows_v, *sems):
        gsems = sems[:NBUF]
        osems = sems[NBUF:]
        wid = lax.axis_index("s") * nc + lax.axis_index("c")
        srow_base = wid * srows

        pltpu.sync_copy(idx_hbm.at[pl.ds(wid * srows, srows)], idx_v)

        def gather_descs(t):
            bslot = t % NBUF
            return [
                (
                    tab_hbm.at[idx_v.at[t, pl.ds(off, ln)]],
                    rows_v.at[bslot, pl.ds(off, ln)],
                    gsems[bslot],
                )
                for off, ln in splits
            ]

        def write_desc(t):
            bslot = t % NBUF
            return (
                rows_v.at[bslot],
                out_hbm.at[srow_base + t],
                osems[bslot],
            )

        def scale_rows(bslot):
            def body(r, carry):
                r0 = r * rows_per_iter
                for i in range(rows_per_iter):
                    for j in range(d // LANES):
                        sl = (bslot, r0 + i, pl.ds(j * LANES, LANES))
                        rows_v[sl] = rows_v[sl] * SCALE
                return carry

            lax.fori_loop(0, p // rows_per_iter, body, 0, unroll=False)

        for t in range(srows + LAG):
            if t < srows:
                if t >= NBUF:
                    pltpu.make_async_copy(*write_desc(t - NBUF)).wait()
                for desc in gather_descs(t):
                    pltpu.async_copy(*desc)
            tp = t - LAG
            if tp >= 0:
                for desc in gather_descs(tp):
                    pltpu.make_async_copy(*desc).wait()
                scale_rows(tp % NBUF)
                pltpu.async_copy(*write_desc(tp))

        for t in range(srows + LAG - NBUF, srows):
            pltpu.make_async_copy(*write_desc(t)).wait()

    return gather_kernel


def kernel(x, table):
    v, d = table.shape
    s, p = x.shape
    xi = x.astype(jnp.int32)
    return _make_gather(s, p, v, d)(xi, table)
